# baseline (speedup 1.0000x reference)
import functools

import jax
import jax.numpy as jnp
import numpy as np
from jax import lax
from jax.experimental import pallas as pl
from jax.experimental.pallas import tpu as pltpu
from jax.experimental.pallas import tpu_sc as plsc

MAX_LEN = 200
EMB_DIM = 64
BATCH = 4096
N_VOCAB = 1000000

NUM_CORES = 2
NUM_SUBCORES = 16
NUM_WORKERS = NUM_CORES * NUM_SUBCORES

VCHUNK = 128
N_CHUNKS = (N_VOCAB + VCHUNK - 1) // VCHUNK
CHUNKS_PER_W = N_CHUNKS // NUM_WORKERS
CHUNK_REM = N_CHUNKS - CHUNKS_PER_W * NUM_WORKERS

BLOCK = 128
NBLK = BATCH // BLOCK


def _pe_const():
    pos = np.expand_dims(np.arange(MAX_LEN), 1)
    pe = pos / np.power(1000, 2 * np.expand_dims(np.arange(EMB_DIM) // 2, 0) / EMB_DIM)
    pe[:, 0::2] = np.sin(pe[:, 0::2])
    pe[:, 1::2] = np.cos(pe[:, 1::2])
    return pe.astype(np.float32)


_PE = _pe_const()
_IOTA16 = tuple(range(16))


@functools.partial(
    pl.kernel,
    out_type=jax.ShapeDtypeStruct((N_VOCAB, 2 * EMB_DIM), jnp.float32),
    mesh=plsc.VectorSubcoreMesh(core_axis_name="c", subcore_axis_name="s"),
    scratch_types=[
        [pltpu.VMEM((EMB_DIM, VCHUNK), jnp.float32) for _ in range(2)],
        [pltpu.VMEM((VCHUNK, 2 * EMB_DIM), jnp.float32) for _ in range(2)],
        pltpu.SemaphoreType.DMA,
        pltpu.SemaphoreType.DMA,
    ],
    compiler_params=pltpu.CompilerParams(needs_layout_passes=False),
)
def _detile(tabt_hbm, dup_hbm, srcs, rms, isem, osem):
    wid = lax.axis_index("s") * NUM_CORES + lax.axis_index("c")
    base = wid * CHUNKS_PER_W + jnp.minimum(wid, CHUNK_REM)
    cnt = CHUNKS_PER_W + jnp.where(wid < CHUNK_REM, 1, 0)

    iota16 = lax.iota(jnp.int32, 16)
    rows_j = [iota16 + 16 * j for j in range(EMB_DIM // 16)]

    def chunk_in_start(c, src):
        pltpu.async_copy(tabt_hbm.at[:, pl.ds(c * VCHUNK, VCHUNK)], src, isem)

    def chunk_in_wait(src):
        pltpu.make_async_copy(tabt_hbm.at[:, pl.ds(0, VCHUNK)], src, isem).wait()

    def shuffle(src, rm):
        for v in range(VCHUNK):
            cols = iota16 * 0 + v
            for j in range(EMB_DIM // 16):
                val = plsc.load_gather(src, [rows_j[j], cols])
                rm[v, pl.ds(16 * j, 16)] = val
                rm[v, pl.ds(EMB_DIM + 16 * j, 16)] = val

    def out_start(c, rm):
        @pl.when(c < N_CHUNKS - 1)
        def _():
            pltpu.async_copy(rm, dup_hbm.at[pl.ds(c * VCHUNK, VCHUNK), :], osem)

        @pl.when(c == N_CHUNKS - 1)
        def _():
            pltpu.async_copy(
                rm.at[pl.ds(0, VCHUNK // 2), :],
                dup_hbm.at[pl.ds(c * VCHUNK, VCHUNK // 2), :],
                osem,
            )

    def out_wait(c, rm):
        @pl.when(c < N_CHUNKS - 1)
        def _():
            pltpu.make_async_copy(
                rm, dup_hbm.at[pl.ds(0, VCHUNK), :], osem
            ).wait()

        @pl.when(c == N_CHUNKS - 1)
        def _():
            pltpu.make_async_copy(
                rm.at[pl.ds(0, VCHUNK // 2), :],
                dup_hbm.at[pl.ds(0, VCHUNK // 2), :],
                osem,
            ).wait()

    chunk_in_start(base, srcs[0])

    def body(i, carry):
        for p in range(2):
            k = 2 * i + p
            c = base + k

            @pl.when(k < cnt)
            def _():
                @pl.when(k + 1 < cnt)
                def _():
                    chunk_in_start(c + 1, srcs[(p + 1) % 2])

                chunk_in_wait(srcs[p])
                @pl.when(k >= 2)
                def _():
                    out_wait(c - 2, rms[p])

                shuffle(srcs[p], rms[p])
                out_start(c, rms[p])

        return carry

    lax.fori_loop(0, (CHUNKS_PER_W + 2) // 2, body, 0)

    @pl.when(cnt % 2 == 0)
    def _():
        out_wait(base + cnt - 2, rms[0])
        out_wait(base + cnt - 1, rms[1])

    @pl.when(cnt % 2 == 1)
    def _():
        out_wait(base + cnt - 2, rms[1])
        out_wait(base + cnt - 1, rms[0])


@functools.partial(
    pl.kernel,
    out_type=jax.ShapeDtypeStruct((MAX_LEN, EMB_DIM, BATCH), jnp.float32),
    mesh=plsc.VectorSubcoreMesh(core_axis_name="c", subcore_axis_name="s"),
    scratch_types=[
        pltpu.VMEM((BATCH,), jnp.int32),
        pltpu.VMEM((MAX_LEN, EMB_DIM), jnp.float32),
        [pltpu.VMEM((BLOCK, 2 * EMB_DIM), jnp.float32) for _ in range(2)],
        [pltpu.VMEM((EMB_DIM, BLOCK), jnp.float32) for _ in range(2)],
        pltpu.SemaphoreType.DMA,
        pltpu.SemaphoreType.DMA,
    ],
    compiler_params=pltpu.CompilerParams(needs_layout_passes=False),
)
def _gather(xf_hbm, dup_hbm, pe_hbm, out_hbm, idx_v, pe_v, bufs, bufts, gsem, osem):
    wid = lax.axis_index("s") * NUM_CORES + lax.axis_index("c")
    t_lo = wid * MAX_LEN // NUM_WORKERS
    t_hi = (wid + 1) * MAX_LEN // NUM_WORKERS
    pltpu.sync_copy(pe_hbm, pe_v)

    iota16 = lax.iota(jnp.int32, 16)
    rows_g = iota16
    cols_d = [iota16 * 0 + d for d in range(EMB_DIM)]

    def gather_start(bi, buf):
        pltpu.async_copy(dup_hbm.at[idx_v.at[pl.ds(bi * BLOCK, BLOCK)]], buf, gsem)

    def gather_wait(buf):
        pltpu.make_async_copy(
            dup_hbm.at[idx_v.at[pl.ds(0, BLOCK)]], buf, gsem
        ).wait()

    def process(buf, buft, t):
        pe_regs = [pe_v[t, pl.ds(16 * j, 16)] for j in range(EMB_DIM // 16)]

        def pe_body(r8, c):
            for u in range(8):
                r = r8 * 8 + u
                for j in range(EMB_DIM // 16):
                    plsc.addupdate(buf.at[r, pl.ds(16 * j, 16)], pe_regs[j])
                    plsc.addupdate(
                        buf.at[r, pl.ds(EMB_DIM + 16 * j, 16)], pe_regs[j]
                    )
            return c

        lax.fori_loop(0, BLOCK // 8, pe_body, 0)

        def tr_body(g, c):
            rows = rows_g + g * 16
            for d in range(EMB_DIM):
                buft[d, pl.ds(g * 16, 16)] = plsc.load_gather(
                    buf, [rows, cols_d[d]]
                )
            return c

        lax.fori_loop(0, BLOCK // 16, tr_body, 0)

    def out_start(bi, buft, t):
        pltpu.async_copy(buft, out_hbm.at[t, :, pl.ds(bi * BLOCK, BLOCK)], osem)

    def out_wait(buft, t):
        pltpu.make_async_copy(
            buft, out_hbm.at[t, :, pl.ds(0, BLOCK)], osem
        ).wait()

    def t_body(t, carry):
        pltpu.sync_copy(xf_hbm.at[pl.ds(t * BATCH, BATCH)], idx_v)
        gather_start(0, bufs[0])

        def blk_body(i, c):
            for p in range(2):
                bi = 2 * i + p

                @pl.when(bi + 1 < NBLK)
                def _():
                    gather_start(bi + 1, bufs[(p + 1) % 2])

                gather_wait(bufs[p])

                @pl.when(bi >= 2)
                def _():
                    out_wait(bufts[p], t)

                process(bufs[p], bufts[p], t)
                out_start(bi, bufts[p], t)
            return c

        lax.fori_loop(0, NBLK // 2, blk_body, 0)
        out_wait(bufts[0], t)
        out_wait(bufts[1], t)
        return carry

    lax.fori_loop(t_lo, t_hi, t_body, 0)


def kernel(x, table):
    xf = jnp.transpose(x).reshape(-1).astype(jnp.int32)
    tabt = jnp.transpose(table)
    dup = _detile(tabt)
    out3 = _gather(xf, dup, jnp.asarray(_PE))
    return jnp.transpose(out3, (2, 0, 1))


# speedup vs baseline: 2.8523x; 2.8523x over previous
import functools

import jax
import jax.numpy as jnp
import numpy as np
from jax import lax
from jax.experimental import pallas as pl
from jax.experimental.pallas import tpu as pltpu
from jax.experimental.pallas import tpu_sc as plsc

MAX_LEN = 200
EMB_DIM = 64
BATCH = 4096
N_VOCAB = 1000000

NUM_CORES = 2
NUM_SUBCORES = 16
NUM_WORKERS = NUM_CORES * NUM_SUBCORES

BLOCK = 128
NBLK = BATCH // BLOCK


def _pe_const():
    pos = np.expand_dims(np.arange(MAX_LEN), 1)
    pe = pos / np.power(1000, 2 * np.expand_dims(np.arange(EMB_DIM) // 2, 0) / EMB_DIM)
    pe[:, 0::2] = np.sin(pe[:, 0::2])
    pe[:, 1::2] = np.cos(pe[:, 1::2])
    return pe.astype(np.float32)


_PE = _pe_const()


@functools.partial(
    pl.kernel,
    out_type=jax.ShapeDtypeStruct((MAX_LEN, EMB_DIM, BATCH), jnp.float32),
    mesh=plsc.VectorSubcoreMesh(core_axis_name="c", subcore_axis_name="s"),
    scratch_types=[
        pltpu.VMEM((BATCH,), jnp.int32),
        pltpu.VMEM((MAX_LEN, EMB_DIM), jnp.float32),
        [pltpu.VMEM((BLOCK, 2 * EMB_DIM), jnp.float32) for _ in range(2)],
        [pltpu.VMEM((EMB_DIM, BLOCK), jnp.float32) for _ in range(2)],
        pltpu.SemaphoreType.DMA,
        pltpu.SemaphoreType.DMA,
    ],
    compiler_params=pltpu.CompilerParams(needs_layout_passes=False),
)
def _gather(xf_hbm, tab_hbm, pe_hbm, out_hbm, idx_v, pe_v, bufs, bufts, gsem, osem):
    wid = lax.axis_index("s") * NUM_CORES + lax.axis_index("c")
    t_lo = wid * MAX_LEN // NUM_WORKERS
    t_hi = (wid + 1) * MAX_LEN // NUM_WORKERS
    pltpu.sync_copy(pe_hbm, pe_v)

    iota16 = lax.iota(jnp.int32, 16)
    diag = [(iota16 + k) % 16 for k in range(16)]

    def gather_start(bi, buf):
        pltpu.async_copy(tab_hbm.at[idx_v.at[pl.ds(bi * BLOCK, BLOCK)]], buf, gsem)

    def gather_wait(buf):
        pltpu.make_async_copy(
            tab_hbm.at[idx_v.at[pl.ds(0, BLOCK)]], buf, gsem
        ).wait()

    def process(buf, buft, t):
        pe_regs = [pe_v[t, pl.ds(16 * j, 16)] for j in range(EMB_DIM // 16)]

        def pe_body(r8, c):
            for u in range(8):
                r = r8 * 8 + u
                for j in range(EMB_DIM // 16):
                    plsc.addupdate(buf.at[r, pl.ds(16 * j, 16)], pe_regs[j])
            return c

        lax.fori_loop(0, BLOCK // 8, pe_body, 0)

        def k_body(k, c):
            dk = (iota16 + k) % 16
            dcols = [dk + 16 * j for j in range(EMB_DIM // 16)]

            def g_body(g, c2):
                rows = iota16 + 16 * g
                for j in range(EMB_DIM // 16):
                    val = plsc.load_gather(buf, [rows, dcols[j]])
                    plsc.store_scatter(buft, [dcols[j], rows], val)
                return c2

            lax.fori_loop(0, BLOCK // 16, g_body, 0)
            return c

        lax.fori_loop(0, 16, k_body, 0)

    def out_start(bi, buft, t):
        pltpu.async_copy(buft, out_hbm.at[t, :, pl.ds(bi * BLOCK, BLOCK)], osem)

    def out_wait(buft, t):
        pltpu.make_async_copy(
            buft, out_hbm.at[t, :, pl.ds(0, BLOCK)], osem
        ).wait()

    def t_body(t, carry):
        pltpu.sync_copy(xf_hbm.at[pl.ds(t * BATCH, BATCH)], idx_v)
        gather_start(0, bufs[0])

        def blk_body(i, c):
            for p in range(2):
                bi = 2 * i + p

                @pl.when(bi + 1 < NBLK)
                def _():
                    gather_start(bi + 1, bufs[(p + 1) % 2])

                gather_wait(bufs[p])

                @pl.when(bi >= 2)
                def _():
                    out_wait(bufts[p], t)

                process(bufs[p], bufts[p], t)
                out_start(bi, bufts[p], t)
            return c

        lax.fori_loop(0, NBLK // 2, blk_body, 0)
        out_wait(bufts[0], t)
        out_wait(bufts[1], t)
        return carry

    lax.fori_loop(t_lo, t_hi, t_body, 0)


def kernel(x, table):
    xf = jnp.transpose(x).reshape(-1).astype(jnp.int32)
    tab128 = jnp.pad(table, ((0, 0), (0, EMB_DIM)))
    out3 = _gather(xf, tab128, jnp.asarray(_PE))
    return jnp.transpose(out3, (2, 0, 1))


# speedup vs baseline: 2.9404x; 1.0309x over previous
import functools

import jax
import jax.numpy as jnp
import numpy as np
from jax import lax
from jax.experimental import pallas as pl
from jax.experimental.pallas import tpu as pltpu
from jax.experimental.pallas import tpu_sc as plsc

MAX_LEN = 200
EMB_DIM = 64
BATCH = 4096
N_VOCAB = 1000000

NUM_CORES = 2
NUM_SUBCORES = 16
NUM_WORKERS = NUM_CORES * NUM_SUBCORES

BLOCK = 128
NBLK = BATCH // BLOCK


def _pe_const():
    pos = np.expand_dims(np.arange(MAX_LEN), 1)
    pe = pos / np.power(1000, 2 * np.expand_dims(np.arange(EMB_DIM) // 2, 0) / EMB_DIM)
    pe[:, 0::2] = np.sin(pe[:, 0::2])
    pe[:, 1::2] = np.cos(pe[:, 1::2])
    return pe.astype(np.float32)


_PE = _pe_const()


@functools.partial(
    pl.kernel,
    out_type=jax.ShapeDtypeStruct((MAX_LEN, EMB_DIM, BATCH), jnp.float32),
    mesh=plsc.VectorSubcoreMesh(core_axis_name="c", subcore_axis_name="s"),
    scratch_types=[
        pltpu.VMEM((BATCH,), jnp.int32),
        pltpu.VMEM((MAX_LEN, EMB_DIM), jnp.float32),
        [pltpu.VMEM((BLOCK, 2 * EMB_DIM), jnp.float32) for _ in range(2)],
        [pltpu.VMEM((EMB_DIM, BLOCK), jnp.float32) for _ in range(2)],
        pltpu.SemaphoreType.DMA,
        pltpu.SemaphoreType.DMA,
    ],
    compiler_params=pltpu.CompilerParams(needs_layout_passes=False),
)
def _gather(xf_hbm, tab_hbm, pe_hbm, out_hbm, idx_v, pe_v, bufs, bufts, gsem, osem):
    wid = lax.axis_index("s") * NUM_CORES + lax.axis_index("c")
    t_lo = wid * MAX_LEN // NUM_WORKERS
    t_hi = (wid + 1) * MAX_LEN // NUM_WORKERS
    pltpu.sync_copy(pe_hbm, pe_v)

    iota16 = lax.iota(jnp.int32, 16)
    diag = [(iota16 + k) % 16 for k in range(16)]

    def gather_start(bi, buf):
        pltpu.async_copy(tab_hbm.at[idx_v.at[pl.ds(bi * BLOCK, BLOCK)]], buf, gsem)

    def gather_wait(buf):
        pltpu.make_async_copy(
            tab_hbm.at[idx_v.at[pl.ds(0, BLOCK)]], buf, gsem
        ).wait()

    def process(buf, buft, t):
        pe_regs = [pe_v[t, pl.ds(16 * j, 16)] for j in range(EMB_DIM // 16)]

        def pe_body(r16, c):
            for u in range(16):
                r = r16 * 16 + u
                for j in range(EMB_DIM // 16):
                    plsc.addupdate(buf.at[r, pl.ds(16 * j, 16)], pe_regs[j])
            return c

        lax.fori_loop(0, BLOCK // 16, pe_body, 0)

        rows_g = [iota16 + 16 * g for g in range(BLOCK // 16)]

        def k_body(k, c):
            dk = (iota16 + k) % 16
            for j in range(EMB_DIM // 16):
                dcols = dk + 16 * j
                for g in range(BLOCK // 16):
                    val = plsc.load_gather(buf, [rows_g[g], dcols])
                    plsc.store_scatter(buft, [dcols, rows_g[g]], val)
            return c

        lax.fori_loop(0, 16, k_body, 0)

    def out_start(bi, buft, t):
        pltpu.async_copy(buft, out_hbm.at[t, :, pl.ds(bi * BLOCK, BLOCK)], osem)

    def out_wait(buft, t):
        pltpu.make_async_copy(
            buft, out_hbm.at[t, :, pl.ds(0, BLOCK)], osem
        ).wait()

    def t_body(t, carry):
        pltpu.sync_copy(xf_hbm.at[pl.ds(t * BATCH, BATCH)], idx_v)
        gather_start(0, bufs[0])

        def blk_body(i, c):
            for p in range(2):
                bi = 2 * i + p

                @pl.when(bi + 1 < NBLK)
                def _():
                    gather_start(bi + 1, bufs[(p + 1) % 2])

                gather_wait(bufs[p])

                @pl.when(bi >= 2)
                def _():
                    out_wait(bufts[p], t)

                process(bufs[p], bufts[p], t)
                out_start(bi, bufts[p], t)
            return c

        lax.fori_loop(0, NBLK // 2, blk_body, 0)
        out_wait(bufts[0], t)
        out_wait(bufts[1], t)
        return carry

    lax.fori_loop(t_lo, t_hi, t_body, 0)


def kernel(x, table):
    xf = jnp.transpose(x).reshape(-1).astype(jnp.int32)
    tab128 = jnp.pad(table, ((0, 0), (0, EMB_DIM)))
    out3 = _gather(xf, tab128, jnp.asarray(_PE))
    return jnp.transpose(out3, (2, 0, 1))
